# baseline (speedup 1.0000x reference)
import functools
import numpy as np
import jax
import jax.numpy as jnp
from jax import lax
from jax.experimental import pallas as pl
from jax.experimental.pallas import tpu as pltpu, tpu_sc as plsc

B, H, W, C = 4, 512, 512, 16
NX, NY = 74, 74
GX, GY = NX - 3, NY - 3
SX = float(W) / GX
SY = float(W) / GY
NPIX = B * H * W


def _basis_matrix(n_pix, s, n_ctrl):
    t = (np.linspace(0.0, n_pix - 1.0, n_pix).astype(np.float32)
         / np.float32(s)).astype(np.float32)
    pf = np.floor(t)
    u = (t - pf).astype(np.float32)
    u2, u3 = u * u, u * u * u
    U = [(-u3 + 3 * u2 - 3 * u + 1) / 6,
         (3 * u3 - 6 * u2 + 4) / 6,
         (-u3 * 3 + 3 * u2 + 3 * u + 1) / 6,
         u3 / 6]
    A = np.zeros((n_pix, n_ctrl), np.float32)
    p = pf.astype(np.int32)
    rows = np.arange(n_pix)
    for i, Ui in enumerate(U):
        A[rows, p + i] = Ui.astype(np.float32)
    return A


_AX_T = _basis_matrix(W, SX, NX).T
_AY = _basis_matrix(H, SY, NY)


def _delta_body(theta_ref, axt_ref, ay_ref, out_ref):
    t = theta_ref[0, 0]
    tmp = jnp.dot(t, axt_ref[...], preferred_element_type=jnp.float32,
                  precision=lax.Precision.HIGHEST)
    out_ref[0, 0] = jnp.dot(ay_ref[...], tmp,
                            preferred_element_type=jnp.float32,
                            precision=lax.Precision.HIGHEST)


def _compute_delta(theta):
    axt = jnp.asarray(_AX_T)
    ay = jnp.asarray(_AY)
    return pl.pallas_call(
        _delta_body,
        grid=(2, B),
        in_specs=[
            pl.BlockSpec((1, 1, NY, NX), lambda c, b: (b, c, 0, 0)),
            pl.BlockSpec((NX, W), lambda c, b: (0, 0)),
            pl.BlockSpec((H, NY), lambda c, b: (0, 0)),
        ],
        out_specs=pl.BlockSpec((1, 1, H, W), lambda c, b: (c, b, 0, 0)),
        out_shape=jax.ShapeDtypeStruct((2, B, H, W), jnp.float32),
    )(theta, axt, ay)


_INFO = plsc.get_sparse_core_info()
_NC, _NS = _INFO.num_cores, _INFO.num_subcores
_NW = _NC * _NS
_PP = NPIX // _NW
_CH = 128
_NCHUNK = _PP // _CH


def _sample_body(fmap_hbm, dx_hbm, dy_hbm, out_hbm,
                 dx_v, dy_v, i00_v, i10_v, i01_v, i11_v,
                 wa_v, wb_v, wc_v, wd_v,
                 r00_v, r10_v, r01_v, r11_v, out_v, sem):
    wid = lax.axis_index("s") * _NC + lax.axis_index("c")
    base = wid * _PP
    lanes = lax.iota(jnp.int32, 16)

    def chunk_body(ci, carry):
        p0 = base + ci * _CH
        pltpu.sync_copy(dx_hbm.at[pl.ds(p0, _CH)], dx_v)
        pltpu.sync_copy(dy_hbm.at[pl.ds(p0, _CH)], dy_v)
        for i in range(_CH // 16):
            sl = pl.ds(i * 16, 16)
            pv = p0 + i * 16 + lanes
            dx = dx_v[sl]
            dy = dy_v[sl]
            wcol = jnp.bitwise_and(pv, 511)
            hrow = jnp.bitwise_and(lax.shift_right_logical(pv, 9), 511)
            bidx = lax.shift_right_logical(pv, 18)
            x = wcol.astype(jnp.float32) + dx
            y = hrow.astype(jnp.float32) + dy
            xt = x.astype(jnp.int32)
            x0 = jnp.where(xt.astype(jnp.float32) > x, xt - 1, xt)
            yt = y.astype(jnp.int32)
            y0 = jnp.where(yt.astype(jnp.float32) > y, yt - 1, yt)
            x0c = jnp.clip(x0, 0, W - 1)
            x1c = jnp.clip(x0 + 1, 0, W - 1)
            y0c = jnp.clip(y0, 0, H - 1)
            y1c = jnp.clip(y0 + 1, 0, H - 1)
            x0f = x0c.astype(jnp.float32)
            x1f = x1c.astype(jnp.float32)
            y0f = y0c.astype(jnp.float32)
            y1f = y1c.astype(jnp.float32)
            wa_v[sl] = (x1f - x) * (y1f - y)
            wb_v[sl] = (x1f - x) * (y - y0f)
            wc_v[sl] = (x - x0f) * (y1f - y)
            wd_v[sl] = (x - x0f) * (y - y0f)
            bb = lax.shift_left(bidx, 18)
            ry0 = bb + lax.shift_left(y0c, 9)
            ry1 = bb + lax.shift_left(y1c, 9)
            i00_v[sl] = ry0 + x0c
            i10_v[sl] = ry1 + x0c
            i01_v[sl] = ry0 + x1c
            i11_v[sl] = ry1 + x1c
        cps = [pltpu.async_copy(fmap_hbm.at[iv], rv, sem)
               for iv, rv in ((i00_v, r00_v), (i10_v, r10_v),
                              (i01_v, r01_v), (i11_v, r11_v))]
        for cp in cps:
            cp.wait()

        def px(p, c2):
            fa = plsc.load_gather(wa_v, [jnp.full((16,), p, jnp.int32)])
            fb = plsc.load_gather(wb_v, [jnp.full((16,), p, jnp.int32)])
            fc = plsc.load_gather(wc_v, [jnp.full((16,), p, jnp.int32)])
            fd = plsc.load_gather(wd_v, [jnp.full((16,), p, jnp.int32)])
            out_v[p] = (fa * r00_v[p] + fb * r10_v[p]
                        + fc * r01_v[p] + fd * r11_v[p])
            return c2
        lax.fori_loop(0, _CH, px, 0, unroll=4)
        pltpu.sync_copy(out_v, out_hbm.at[pl.ds(p0, _CH)])
        return carry

    lax.fori_loop(0, _NCHUNK, chunk_body, 0)


def _sample(fmap2d, dx, dy):
    mesh = plsc.VectorSubcoreMesh(core_axis_name="c", subcore_axis_name="s")
    f = functools.partial(
        pl.kernel,
        mesh=mesh,
        compiler_params=pltpu.CompilerParams(use_tc_tiling_on_sc=False,
                                             needs_layout_passes=False),
        out_type=jax.ShapeDtypeStruct((NPIX, C), jnp.float32),
        scratch_types=[
            pltpu.VMEM((_CH,), jnp.float32),
            pltpu.VMEM((_CH,), jnp.float32),
            pltpu.VMEM((_CH,), jnp.int32),
            pltpu.VMEM((_CH,), jnp.int32),
            pltpu.VMEM((_CH,), jnp.int32),
            pltpu.VMEM((_CH,), jnp.int32),
            pltpu.VMEM((_CH,), jnp.float32),
            pltpu.VMEM((_CH,), jnp.float32),
            pltpu.VMEM((_CH,), jnp.float32),
            pltpu.VMEM((_CH,), jnp.float32),
            pltpu.VMEM((_CH, C), jnp.float32),
            pltpu.VMEM((_CH, C), jnp.float32),
            pltpu.VMEM((_CH, C), jnp.float32),
            pltpu.VMEM((_CH, C), jnp.float32),
            pltpu.VMEM((_CH, C), jnp.float32),
            pltpu.SemaphoreType.DMA,
        ],
    )(_sample_body)
    return f(fmap2d, dx, dy)


def kernel(input_fmap, theta):
    delta = _compute_delta(theta)
    fmap2d = input_fmap.reshape(NPIX, C)
    dx = delta[0].reshape(NPIX)
    dy = delta[1].reshape(NPIX)
    out2d = _sample(fmap2d, dx, dy)
    return out2d.reshape(B, H, W, C), delta


# speedup vs baseline: 1.5005x; 1.5005x over previous
import functools
import numpy as np
import jax
import jax.numpy as jnp
from jax import lax
from jax.experimental import pallas as pl
from jax.experimental.pallas import tpu as pltpu, tpu_sc as plsc

B, H, W, C = 4, 512, 512, 16
NX, NY = 74, 74
GX, GY = NX - 3, NY - 3
SX = float(W) / GX
SY = float(W) / GY
NPIX = B * H * W


def _basis_matrix(n_pix, s, n_ctrl):
    t = (np.linspace(0.0, n_pix - 1.0, n_pix).astype(np.float32)
         / np.float32(s)).astype(np.float32)
    pf = np.floor(t)
    u = (t - pf).astype(np.float32)
    u2, u3 = u * u, u * u * u
    U = [(-u3 + 3 * u2 - 3 * u + 1) / 6,
         (3 * u3 - 6 * u2 + 4) / 6,
         (-u3 * 3 + 3 * u2 + 3 * u + 1) / 6,
         u3 / 6]
    A = np.zeros((n_pix, n_ctrl), np.float32)
    p = pf.astype(np.int32)
    rows = np.arange(n_pix)
    for i, Ui in enumerate(U):
        A[rows, p + i] = Ui.astype(np.float32)
    return A


_AX_T = _basis_matrix(W, SX, NX).T
_AY = _basis_matrix(H, SY, NY)


def _delta_body(theta_ref, axt_ref, ay_ref, out_ref):
    t = theta_ref[0, 0]
    tmp = jnp.dot(t, axt_ref[...], preferred_element_type=jnp.float32,
                  precision=lax.Precision.HIGHEST)
    out_ref[0, 0] = jnp.dot(ay_ref[...], tmp,
                            preferred_element_type=jnp.float32,
                            precision=lax.Precision.HIGHEST)


def _compute_delta(theta):
    axt = jnp.asarray(_AX_T)
    ay = jnp.asarray(_AY)
    return pl.pallas_call(
        _delta_body,
        grid=(2, B),
        in_specs=[
            pl.BlockSpec((1, 1, NY, NX), lambda c, b: (b, c, 0, 0)),
            pl.BlockSpec((NX, W), lambda c, b: (0, 0)),
            pl.BlockSpec((H, NY), lambda c, b: (0, 0)),
        ],
        out_specs=pl.BlockSpec((1, 1, H, W), lambda c, b: (c, b, 0, 0)),
        out_shape=jax.ShapeDtypeStruct((2, B, H, W), jnp.float32),
    )(theta, axt, ay)


_INFO = plsc.get_sparse_core_info()
_NC, _NS = _INFO.num_cores, _INFO.num_subcores
_NW = _NC * _NS
_PP = NPIX // _NW
_CH = 512
_NG = 128
_NSPLIT = _CH // _NG
_NCHUNK = _PP // _CH


def _sample_body(fmap_hbm, dx_hbm, dy_hbm, out_hbm,
                 dxy0, dxy1, idx0, idx1, w0, w1,
                 rows0, rows1, outv0, outv1,
                 sdxy0, sdxy1, sg0, sg1, so0, so1):
    wid = lax.axis_index("s") * _NC + lax.axis_index("c")
    base = wid * _PP
    lanes = lax.iota(jnp.int32, 16)

    dxy = (dxy0, dxy1)
    idx = (idx0, idx1)
    wv = (w0, w1)
    rows = (rows0, rows1)
    outv = (outv0, outv1)
    sdxy = (sdxy0, sdxy1)
    sg = (sg0, sg1)
    so = (so0, so1)

    def fire_dxy(ci, k):
        sl = pl.ds(base + ci * _CH, _CH)
        pltpu.make_async_copy(dx_hbm.at[sl], dxy[k].at[0], sdxy[k]).start()
        pltpu.make_async_copy(dy_hbm.at[sl], dxy[k].at[1], sdxy[k]).start()

    def wait_dxy(k):
        sl = pl.ds(base, _CH)
        pltpu.make_async_copy(dx_hbm.at[sl], dxy[k].at[0], sdxy[k]).wait()
        pltpu.make_async_copy(dy_hbm.at[sl], dxy[k].at[1], sdxy[k]).wait()

    def compute_idxw(ci, k):
        p0 = base + ci * _CH

        def grp(i, carry):
            sl = pl.ds(i * 16, 16)
            pv = p0 + i * 16 + lanes
            dx = dxy[k][0, sl]
            dy = dxy[k][1, sl]
            wcol = jnp.bitwise_and(pv, 511)
            hrow = jnp.bitwise_and(lax.shift_right_logical(pv, 9), 511)
            bidx = lax.shift_right_logical(pv, 18)
            x = wcol.astype(jnp.float32) + dx
            y = hrow.astype(jnp.float32) + dy
            xt = x.astype(jnp.int32)
            x0 = jnp.where(xt.astype(jnp.float32) > x, xt - 1, xt)
            yt = y.astype(jnp.int32)
            y0 = jnp.where(yt.astype(jnp.float32) > y, yt - 1, yt)
            x0c = jnp.clip(x0, 0, W - 1)
            x1c = jnp.clip(x0 + 1, 0, W - 1)
            y0c = jnp.clip(y0, 0, H - 1)
            y1c = jnp.clip(y0 + 1, 0, H - 1)
            x0f = x0c.astype(jnp.float32)
            x1f = x1c.astype(jnp.float32)
            y0f = y0c.astype(jnp.float32)
            y1f = y1c.astype(jnp.float32)
            wv[k][0, sl] = (x1f - x) * (y1f - y)
            wv[k][1, sl] = (x1f - x) * (y - y0f)
            wv[k][2, sl] = (x - x0f) * (y1f - y)
            wv[k][3, sl] = (x - x0f) * (y - y0f)
            bb = lax.shift_left(bidx, 18)
            ry0 = bb + lax.shift_left(y0c, 9)
            ry1 = bb + lax.shift_left(y1c, 9)
            idx[k][0, sl] = ry0 + x0c
            idx[k][1, sl] = ry1 + x0c
            idx[k][2, sl] = ry0 + x1c
            idx[k][3, sl] = ry1 + x1c
            return carry

        lax.fori_loop(0, _CH // 16, grp, 0, unroll=4)

    def fire_g(k):
        for j in range(4):
            for s in range(_NSPLIT):
                sl = pl.ds(s * _NG, _NG)
                pltpu.make_async_copy(
                    fmap_hbm.at[idx[k].at[j, sl]],
                    rows[k].at[j, sl], sg[k]).start()

    def wait_g(k):
        for j in range(4):
            for s in range(_NSPLIT):
                sl = pl.ds(s * _NG, _NG)
                pltpu.make_async_copy(
                    fmap_hbm.at[idx[k].at[j, sl]],
                    rows[k].at[j, sl], sg[k]).wait()

    def combine(k):
        def px(p, carry):
            pf = jnp.full((16,), p, jnp.int32)
            fa = plsc.load_gather(wv[k], [jnp.zeros((16,), jnp.int32), pf])
            fb = plsc.load_gather(wv[k], [jnp.full((16,), 1, jnp.int32), pf])
            fc = plsc.load_gather(wv[k], [jnp.full((16,), 2, jnp.int32), pf])
            fd = plsc.load_gather(wv[k], [jnp.full((16,), 3, jnp.int32), pf])
            outv[k][p] = (fa * rows[k][0, p] + fb * rows[k][1, p]
                          + fc * rows[k][2, p] + fd * rows[k][3, p])
            return carry

        lax.fori_loop(0, _CH, px, 0, unroll=4)

    def fire_out(ci, k):
        sl = pl.ds(base + ci * _CH, _CH)
        pltpu.make_async_copy(outv[k], out_hbm.at[sl], so[k]).start()

    def wait_out(k):
        sl = pl.ds(base, _CH)
        pltpu.make_async_copy(outv[k], out_hbm.at[sl], so[k]).wait()

    fire_dxy(0, 0)
    wait_dxy(0)
    compute_idxw(0, 0)
    fire_g(0)
    fire_dxy(1, 1)

    def phase(it, c, k):
        nxt = 1 - k
        wait_dxy(nxt)
        compute_idxw(c + 1, nxt)
        fire_g(nxt)

        @pl.when(c + 2 <= _NCHUNK - 1)
        def _():
            fire_dxy(c + 2, k)

        wait_g(k)

        @pl.when(c >= 2)
        def _():
            wait_out(k)

        combine(k)
        fire_out(c, k)

    def body(it, carry):
        c = 2 * it
        phase(it, c, 0)

        @pl.when(c + 2 <= _NCHUNK - 1)
        def _():
            phase(it, c + 1, 1)

        return carry

    lax.fori_loop(0, _NCHUNK // 2, body, 0)

    wait_g(1)
    wait_out(1)
    combine(1)
    fire_out(_NCHUNK - 1, 1)
    wait_out(0)
    wait_out(1)


def _sample(fmap2d, dx, dy):
    mesh = plsc.VectorSubcoreMesh(core_axis_name="c", subcore_axis_name="s")
    f = functools.partial(
        pl.kernel,
        mesh=mesh,
        compiler_params=pltpu.CompilerParams(use_tc_tiling_on_sc=False,
                                             needs_layout_passes=False),
        out_type=jax.ShapeDtypeStruct((NPIX, C), jnp.float32),
        scratch_types=[
            pltpu.VMEM((2, _CH), jnp.float32),
            pltpu.VMEM((2, _CH), jnp.float32),
            pltpu.VMEM((4, _CH), jnp.int32),
            pltpu.VMEM((4, _CH), jnp.int32),
            pltpu.VMEM((4, _CH), jnp.float32),
            pltpu.VMEM((4, _CH), jnp.float32),
            pltpu.VMEM((4, _CH, C), jnp.float32),
            pltpu.VMEM((4, _CH, C), jnp.float32),
            pltpu.VMEM((_CH, C), jnp.float32),
            pltpu.VMEM((_CH, C), jnp.float32),
            pltpu.SemaphoreType.DMA,
            pltpu.SemaphoreType.DMA,
            pltpu.SemaphoreType.DMA,
            pltpu.SemaphoreType.DMA,
            pltpu.SemaphoreType.DMA,
            pltpu.SemaphoreType.DMA,
        ],
    )(_sample_body)
    return f(fmap2d, dx, dy)


def kernel(input_fmap, theta):
    delta = _compute_delta(theta)
    fmap2d = input_fmap.reshape(NPIX, C)
    dx = delta[0].reshape(NPIX)
    dy = delta[1].reshape(NPIX)
    out2d = _sample(fmap2d, dx, dy)
    return out2d.reshape(B, H, W, C), delta
